# Initial kernel scaffold; baseline (speedup 1.0000x reference)
import jax
import jax.numpy as jnp
from jax.experimental import pallas as pl


def kernel(x, adj, edge, params):
    raise NotImplementedError("write your pallas kernel here")


# speedup vs baseline: 2.2427x; 2.2427x over previous
import functools

import jax
import jax.numpy as jnp
from jax import lax
from jax.experimental import pallas as pl
from jax.experimental.pallas import tpu as pltpu
from jax.experimental.pallas import tpu_sc as plsc

N = 10000
E = 320000
X_DIM = 128
EDGE_DIM = 16
HIDDEN = 128
LATENT = 64
AA_DIM = 20
SS_DIM = 7
X_CLASS = 2
EDGE_CLASS = 3
MAX_SIZE = 30
FX_DIM = AA_DIM * SS_DIM + X_DIM - X_CLASS
FE_DIM = MAX_SIZE * (EDGE_DIM + EDGE_CLASS - 1)

NC = 2
NS = 16
NW = NC * NS
EP = E // NW
C = 80
NCH = EP // C
NP = 10240
RPT = NP // NS

_mesh = plsc.VectorSubcoreMesh(
    core_axis_name="c", subcore_axis_name="s", num_cores=NC, num_subcores=NS)


def _sc_pass_body(hm, gate, src, dst, accp,
                  idx_s, idx_d, rows, gate_v, msg_v, zb, acc_sh, gsem):
    c = lax.axis_index("c")
    s = lax.axis_index("s")
    w = s * NC + c

    zeros = jnp.zeros((16,), jnp.float32)

    for i in range(16):
        for j in range(HIDDEN // 16):
            zb[i, pl.ds(j * 16, 16)] = zeros

    @pl.loop(0, RPT // 16)
    def _zacc(t):
        pltpu.sync_copy(zb, acc_sh.at[pl.ds(s * RPT + t * 16, 16)])

    plsc.subcore_barrier()

    @pl.loop(0, NCH)
    def _chunk(i):
        base = w * EP + i * C
        pltpu.sync_copy(src.at[pl.ds(base, C)], idx_s)
        pltpu.sync_copy(dst.at[pl.ds(base, C)], idx_d)
        pltpu.async_copy(hm.at[idx_s], rows, gsem).wait()
        pltpu.sync_copy(gate.at[pl.ds(base, C)], gate_v)

        def _mrow(r, cy):
            for k2 in range(HIDDEN // 16):
                sl = pl.ds(k2 * 16, 16)
                msg_v[r, sl] = rows[r, sl] * gate_v[r, sl]
            return cy
        lax.fori_loop(0, C, _mrow, 0)

        pltpu.sync_copy(msg_v, acc_sh.at[idx_d], add=True)

    plsc.subcore_barrier()

    idx_o = c * NS + s
    for k in range(RPT // C):
        pltpu.sync_copy(acc_sh.at[pl.ds(s * RPT + k * C, C)], msg_v)
        pltpu.sync_copy(msg_v, accp.at[idx_o, pl.ds(k * C, C)])


def _sc_pass(hm, gate, src, dst):
    out_type = jax.ShapeDtypeStruct((NW, RPT, HIDDEN), jnp.float32)
    scratch = [
        pltpu.VMEM((C,), jnp.int32),
        pltpu.VMEM((C,), jnp.int32),
        pltpu.VMEM((C, HIDDEN), jnp.float32),
        pltpu.VMEM((C, HIDDEN), jnp.float32),
        pltpu.VMEM((C, HIDDEN), jnp.float32),
        pltpu.VMEM((16, HIDDEN), jnp.float32),
        pltpu.VMEM_SHARED((NP, HIDDEN), jnp.float32),
        pltpu.SemaphoreType.DMA,
    ]
    fn = pl.kernel(
        _sc_pass_body,
        out_type=out_type,
        mesh=_mesh,
        scratch_types=scratch,
    )
    return fn(hm, gate, src, dst)


def _deg_pass_body(dst, degp, idx_d, ones_m, deg_sh, gsem):
    c = lax.axis_index("c")
    s = lax.axis_index("s")
    w = s * NC + c

    ones = jnp.ones((16,), jnp.float32)
    zeros = jnp.zeros((16,), jnp.float32)

    def _fones(r, carry):
        for j in range(HIDDEN // 16):
            ones_m[r, pl.ds(j * 16, 16)] = zeros
        return carry
    lax.fori_loop(0, C, _fones, 0)

    @pl.loop(0, RPT // C)
    def _zdeg(t):
        pltpu.sync_copy(ones_m, deg_sh.at[pl.ds(s * RPT + t * C, C)])

    def _fones2(r, carry):
        for j in range(HIDDEN // 16):
            ones_m[r, pl.ds(j * 16, 16)] = ones
        return carry
    lax.fori_loop(0, C, _fones2, 0)

    plsc.subcore_barrier()

    @pl.loop(0, NCH)
    def _chunk(i):
        base = w * EP + i * C
        pltpu.sync_copy(dst.at[pl.ds(base, C)], idx_d)
        pltpu.sync_copy(ones_m, deg_sh.at[idx_d], add=True)

    plsc.subcore_barrier()

    idx_o = c * NS + s
    for k in range(RPT // C):
        pltpu.sync_copy(deg_sh.at[pl.ds(s * RPT + k * C, C)], ones_m)
        pltpu.sync_copy(ones_m, degp.at[idx_o, pl.ds(k * C, C)])


def _deg_pass(dst):
    out_type = jax.ShapeDtypeStruct((NW, RPT, HIDDEN), jnp.float32)
    scratch = [
        pltpu.VMEM((C,), jnp.int32),
        pltpu.VMEM((C, HIDDEN), jnp.float32),
        pltpu.VMEM_SHARED((NP, HIDDEN), jnp.float32),
        pltpu.SemaphoreType.DMA,
    ]
    fn = pl.kernel(
        _deg_pass_body,
        out_type=out_type,
        mesh=_mesh,
        scratch_types=scratch,
    )
    return fn(dst)


EB = 2000


def _gates_body(e_ref, *refs):
    (w10, b10, w11, b11, w12, b12, w13, b13,
     w20, b20, w21, b21, w22, b22, w23, b23, g1_ref, g2_ref) = refs
    e = e_ref[...]

    def net(ws):
        g = e
        for wr, br in ws[:-1]:
            g = jnp.maximum(jnp.dot(g, wr[...],
                                    preferred_element_type=jnp.float32)
                            + br[...], 0.0)
        wr, br = ws[-1]
        return jnp.dot(g, wr[...], preferred_element_type=jnp.float32) + br[...]

    g1_ref[...] = net([(w10, b10), (w11, b11), (w12, b12), (w13, b13)])
    g2_ref[...] = net([(w20, b20), (w21, b21), (w22, b22), (w23, b23)])


def _gates(edge, enet1, enet2):
    wspecs = []
    args = []
    for enet in (enet1, enet2):
        for lin in enet:
            wshape = lin["W"].shape
            args.append(lin["W"])
            wspecs.append(pl.BlockSpec(wshape, lambda i: (0, 0)))
            args.append(lin["b"].reshape(1, -1))
            wspecs.append(pl.BlockSpec((1, wshape[1]), lambda i: (0, 0)))
    grid = E // EB
    return pl.pallas_call(
        _gates_body,
        grid=(grid,),
        in_specs=[pl.BlockSpec((EB, EDGE_DIM), lambda i: (i, 0))] + wspecs,
        out_specs=[pl.BlockSpec((EB, HIDDEN), lambda i: (i, 0))] * 2,
        out_shape=[jax.ShapeDtypeStruct((E, HIDDEN), jnp.float32)] * 2,
    )(edge, *args)


NB = 1000


def _matmul_body(h_ref, w_ref, o_ref):
    o_ref[...] = jnp.dot(h_ref[...], w_ref[...],
                         preferred_element_type=jnp.float32)


def _matmul(h, w):
    return pl.pallas_call(
        _matmul_body,
        grid=(N // NB,),
        in_specs=[pl.BlockSpec((NB, HIDDEN), lambda i: (i, 0)),
                  pl.BlockSpec((HIDDEN, HIDDEN), lambda i: (0, 0))],
        out_specs=pl.BlockSpec((NB, HIDDEN), lambda i: (i, 0)),
        out_shape=jax.ShapeDtypeStruct((N, HIDDEN), jnp.float32),
    )(h, w)


def _comb1_body(x_ref, acc_ref, degp_ref, wr_ref, b_ref, wm_ref,
                h1_ref, hm2_ref, degc_ref):
    a = acc_ref[0] + acc_ref[1]
    dc = jnp.maximum(degp_ref[0, :, 0:1] + degp_ref[1, :, 0:1], 1.0)
    agg = a / dc
    h1 = jnp.maximum(
        jnp.dot(x_ref[...], wr_ref[...], preferred_element_type=jnp.float32)
        + agg + b_ref[...], 0.0)
    h1_ref[...] = h1
    hm2_ref[...] = jnp.dot(h1, wm_ref[...], preferred_element_type=jnp.float32)
    degc_ref[...] = dc


def _comb1(x, accp, degp, w_root, b, w_msg2):
    return pl.pallas_call(
        _comb1_body,
        grid=(N // NB,),
        in_specs=[
            pl.BlockSpec((NB, HIDDEN), lambda i: (i, 0)),
            pl.BlockSpec((NC, NB, HIDDEN), lambda i: (0, i, 0)),
            pl.BlockSpec((NC, NB, HIDDEN), lambda i: (0, i, 0)),
            pl.BlockSpec((HIDDEN, HIDDEN), lambda i: (0, 0)),
            pl.BlockSpec((1, HIDDEN), lambda i: (0, 0)),
            pl.BlockSpec((HIDDEN, HIDDEN), lambda i: (0, 0)),
        ],
        out_specs=[
            pl.BlockSpec((NB, HIDDEN), lambda i: (i, 0)),
            pl.BlockSpec((NB, HIDDEN), lambda i: (i, 0)),
            pl.BlockSpec((NB, 1), lambda i: (i, 0)),
        ],
        out_shape=[
            jax.ShapeDtypeStruct((N, HIDDEN), jnp.float32),
            jax.ShapeDtypeStruct((N, HIDDEN), jnp.float32),
            jax.ShapeDtypeStruct((N, 1), jnp.float32),
        ],
    )(x, accp, degp, w_root, b.reshape(1, -1), w_msg2)


def _tail_body(h1_ref, acc_ref, degc_ref, eps_ref,
               wr_ref, b_ref, wmu_ref, bmu_ref, wlv_ref, blv_ref,
               wx0_ref, bx0_ref, wx1_ref, bx1_ref, wfx_ref, bfx_ref,
               we0_ref, be0_ref, we1_ref, be1_ref, wfe_ref, bfe_ref,
               ox_ref, oe_ref, mu_ref, lv_ref):
    a = acc_ref[0] + acc_ref[1]
    agg = a / degc_ref[...]
    h2 = jnp.maximum(
        jnp.dot(h1_ref[...], wr_ref[...], preferred_element_type=jnp.float32)
        + agg + b_ref[...], 0.0)
    mu = jnp.clip(jnp.dot(h2, wmu_ref[...], preferred_element_type=jnp.float32)
                  + bmu_ref[...], -1.0, 1.0)
    lv = jnp.clip(jnp.dot(h2, wlv_ref[...], preferred_element_type=jnp.float32)
                  + blv_ref[...], -1.0, 1.0)
    mu_ref[...] = mu
    lv_ref[...] = lv
    z = mu + jnp.exp(0.5 * lv) * eps_ref[...]

    dx = jnp.maximum(jnp.dot(z, wx0_ref[...],
                             preferred_element_type=jnp.float32) + bx0_ref[...], 0.0)
    dx = jnp.maximum(jnp.dot(dx, wx1_ref[...],
                             preferred_element_type=jnp.float32) + bx1_ref[...], 0.0)
    ox_ref[...] = jnp.dot(dx, wfx_ref[...],
                          preferred_element_type=jnp.float32) + bfx_ref[...]

    de = jnp.maximum(jnp.dot(z, we0_ref[...],
                             preferred_element_type=jnp.float32) + be0_ref[...], 0.0)
    de = jnp.maximum(jnp.dot(de, we1_ref[...],
                             preferred_element_type=jnp.float32) + be1_ref[...], 0.0)
    oe_ref[...] = jnp.dot(de, wfe_ref[...],
                          preferred_element_type=jnp.float32) + bfe_ref[...]


def _tail(h1, accp, degc, eps, params):
    p = params
    dec_x = p["dec_x"]
    dec_e = p["dec_edge"]
    args = [
        p["ecc"][1]["W_root"], p["ecc"][1]["b"].reshape(1, -1),
        p["W_mu"], p["b_mu"].reshape(1, -1),
        p["W_lv"], p["b_lv"].reshape(1, -1),
        dec_x[0]["W"], dec_x[0]["b"].reshape(1, -1),
        dec_x[1]["W"], dec_x[1]["b"].reshape(1, -1),
        p["W_fx"], p["b_fx"].reshape(1, -1),
        dec_e[0]["W"], dec_e[0]["b"].reshape(1, -1),
        dec_e[1]["W"], dec_e[1]["b"].reshape(1, -1),
        p["W_fe"], p["b_fe"].reshape(1, -1),
    ]
    wspecs = [pl.BlockSpec(a.shape, lambda i: (0, 0)) for a in args]
    return pl.pallas_call(
        _tail_body,
        grid=(N // NB,),
        in_specs=[
            pl.BlockSpec((NB, HIDDEN), lambda i: (i, 0)),
            pl.BlockSpec((NC, NB, HIDDEN), lambda i: (0, i, 0)),
            pl.BlockSpec((NB, 1), lambda i: (i, 0)),
            pl.BlockSpec((NB, LATENT), lambda i: (i, 0)),
        ] + wspecs,
        out_specs=[
            pl.BlockSpec((NB, FX_DIM), lambda i: (i, 0)),
            pl.BlockSpec((NB, FE_DIM), lambda i: (i, 0)),
            pl.BlockSpec((NB, LATENT), lambda i: (i, 0)),
            pl.BlockSpec((NB, LATENT), lambda i: (i, 0)),
        ],
        out_shape=[
            jax.ShapeDtypeStruct((N, FX_DIM), jnp.float32),
            jax.ShapeDtypeStruct((N, FE_DIM), jnp.float32),
            jax.ShapeDtypeStruct((N, LATENT), jnp.float32),
            jax.ShapeDtypeStruct((N, LATENT), jnp.float32),
        ],
    )(h1, accp, degc, eps, *args)


def kernel(x, adj, edge, params):
    src = adj[0]
    dst = adj[1]
    eps = jax.random.normal(jax.random.key(42), (N, LATENT), dtype=jnp.float32)

    ecc1, ecc2 = params["ecc"]
    g1, g2 = _gates(edge, ecc1["edge_net"], ecc2["edge_net"])

    hm1 = _matmul(x, ecc1["W_msg"])
    degp = _deg_pass(dst).reshape(NC, NP, HIDDEN)
    accp1 = _sc_pass(hm1, g1, src, dst).reshape(NC, NP, HIDDEN)
    h1, hm2, degc = _comb1(x, accp1, degp, ecc1["W_root"], ecc1["b"],
                           ecc2["W_msg"])
    accp2 = _sc_pass(hm2, g2, src, dst).reshape(NC, NP, HIDDEN)
    out_x, oe, mu, lv = _tail(h1, accp2, degc, eps, params)
    return (out_x, oe.reshape(N, MAX_SIZE, EDGE_DIM + EDGE_CLASS - 1), mu, lv)


# speedup vs baseline: 3.1133x; 1.3882x over previous
import functools

import jax
import jax.numpy as jnp
from jax import lax
from jax.experimental import pallas as pl
from jax.experimental.pallas import tpu as pltpu
from jax.experimental.pallas import tpu_sc as plsc

N = 10000
E = 320000
X_DIM = 128
EDGE_DIM = 16
HIDDEN = 128
LATENT = 64
AA_DIM = 20
SS_DIM = 7
X_CLASS = 2
EDGE_CLASS = 3
MAX_SIZE = 30
FX_DIM = AA_DIM * SS_DIM + X_DIM - X_CLASS
FE_DIM = MAX_SIZE * (EDGE_DIM + EDGE_CLASS - 1)

NC = 2
NS = 16
NW = NC * NS
EP = E // NW
C = 40
NCH = EP // C
CD = 80
NCHD = EP // CD
NP = 10240
RPT = NP // NS

_mesh = plsc.VectorSubcoreMesh(
    core_axis_name="c", subcore_axis_name="s", num_cores=NC, num_subcores=NS)


def _sc_pass_body(hm, gate, src, dst, accp,
                  idx_s0, idx_s1, idx_d0, idx_d1, rows0, rows1, gv0, gv1,
                  zb, acc_sh,
                  isem0, isem1, gsem0, gsem1, lsem0, lsem1):
    idx_s = (idx_s0, idx_s1)
    idx_d = (idx_d0, idx_d1)
    rows = (rows0, rows1)
    gv = (gv0, gv1)
    isem = (isem0, isem1)
    gsem = (gsem0, gsem1)
    lsem = (lsem0, lsem1)
    c = lax.axis_index("c")
    s = lax.axis_index("s")
    w = s * NC + c

    zeros = jnp.zeros((16,), jnp.float32)

    for i in range(16):
        for j in range(HIDDEN // 16):
            zb[i, pl.ds(j * 16, 16)] = zeros

    @pl.loop(0, RPT // 16)
    def _zacc(t):
        pltpu.sync_copy(zb, acc_sh.at[pl.ds(s * RPT + t * 16, 16)])

    plsc.subcore_barrier()

    ebase = w * EP

    def cbase(j):
        return ebase + jnp.where(j < NCH, j, 0) * C

    def start_idx(j, b):
        pltpu.async_copy(src.at[pl.ds(cbase(j), C)], idx_s[b], isem[b])
        pltpu.async_copy(dst.at[pl.ds(cbase(j), C)], idx_d[b], isem[b])

    def wait_idx(b):
        pltpu.make_async_copy(src.at[pl.ds(0, C)], idx_s[b], isem[b]).wait()
        pltpu.make_async_copy(dst.at[pl.ds(0, C)], idx_d[b], isem[b]).wait()

    def start_fetch(j, b):
        pltpu.async_copy(hm.at[idx_s[b]], rows[b], gsem[b])
        pltpu.async_copy(gate.at[pl.ds(cbase(j), C)], gv[b], lsem[b])

    def wait_fetch(b):
        pltpu.make_async_copy(hm.at[idx_s[b]], rows[b], gsem[b]).wait()
        pltpu.make_async_copy(gate.at[pl.ds(0, C)], gv[b], lsem[b]).wait()

    start_idx(0, 0)
    start_idx(1, 1)
    wait_idx(0)
    start_fetch(0, 0)

    @pl.loop(0, NCH, step=2)
    def _chunk(i):
        for b in range(2):
            j = i + b
            o = b ^ 1
            wait_idx(o)
            start_fetch(j + 1, o)
            wait_fetch(b)

            def _mrow(r, cy):
                for k2 in range(HIDDEN // 16):
                    sl = pl.ds(k2 * 16, 16)
                    rows[b][r, sl] = rows[b][r, sl] * gv[b][r, sl]
                return cy
            lax.fori_loop(0, C, _mrow, 0)

            pltpu.sync_copy(rows[b], acc_sh.at[idx_d[b]], add=True)
            start_idx(j + 2, b)

    wait_fetch(0)
    wait_idx(1)

    plsc.subcore_barrier()

    idx_o = c * NS + s
    for k in range(RPT // C):
        b = k % 2
        if k >= 2:
            pltpu.make_async_copy(rows[b], accp.at[idx_o, pl.ds(0, C)],
                                  gsem[b]).wait()
        pltpu.sync_copy(acc_sh.at[pl.ds(s * RPT + k * C, C)], rows[b])
        pltpu.async_copy(rows[b], accp.at[idx_o, pl.ds(k * C, C)], gsem[b])
    pltpu.make_async_copy(rows[0], accp.at[idx_o, pl.ds(0, C)], gsem[0]).wait()
    pltpu.make_async_copy(rows[1], accp.at[idx_o, pl.ds(0, C)], gsem[1]).wait()


def _sc_pass(hm, gate, src, dst):
    out_type = jax.ShapeDtypeStruct((NW, RPT, HIDDEN), jnp.float32)
    scratch = [
        pltpu.VMEM((C,), jnp.int32),
        pltpu.VMEM((C,), jnp.int32),
        pltpu.VMEM((C,), jnp.int32),
        pltpu.VMEM((C,), jnp.int32),
        pltpu.VMEM((C, HIDDEN), jnp.float32),
        pltpu.VMEM((C, HIDDEN), jnp.float32),
        pltpu.VMEM((C, HIDDEN), jnp.float32),
        pltpu.VMEM((C, HIDDEN), jnp.float32),
        pltpu.VMEM((16, HIDDEN), jnp.float32),
        pltpu.VMEM_SHARED((NP, HIDDEN), jnp.float32),
        pltpu.SemaphoreType.DMA,
        pltpu.SemaphoreType.DMA,
        pltpu.SemaphoreType.DMA,
        pltpu.SemaphoreType.DMA,
        pltpu.SemaphoreType.DMA,
        pltpu.SemaphoreType.DMA,
    ]
    fn = pl.kernel(
        _sc_pass_body,
        out_type=out_type,
        mesh=_mesh,
        scratch_types=scratch,
    )
    return fn(hm, gate, src, dst)


def _deg_pass_body(dst, degp, idx_d, ones_m, deg_sh, gsem):
    c = lax.axis_index("c")
    s = lax.axis_index("s")
    w = s * NC + c

    ones = jnp.ones((16,), jnp.float32)
    zeros = jnp.zeros((16,), jnp.float32)

    def _fones(r, carry):
        for j in range(HIDDEN // 16):
            ones_m[r, pl.ds(j * 16, 16)] = zeros
        return carry
    lax.fori_loop(0, CD, _fones, 0)

    @pl.loop(0, RPT // CD)
    def _zdeg(t):
        pltpu.sync_copy(ones_m, deg_sh.at[pl.ds(s * RPT + t * CD, CD)])

    def _fones2(r, carry):
        for j in range(HIDDEN // 16):
            ones_m[r, pl.ds(j * 16, 16)] = ones
        return carry
    lax.fori_loop(0, CD, _fones2, 0)

    plsc.subcore_barrier()

    @pl.loop(0, NCHD)
    def _chunk(i):
        base = w * EP + i * CD
        pltpu.sync_copy(dst.at[pl.ds(base, CD)], idx_d)
        pltpu.sync_copy(ones_m, deg_sh.at[idx_d], add=True)

    plsc.subcore_barrier()

    idx_o = c * NS + s
    for k in range(RPT // CD):
        pltpu.sync_copy(deg_sh.at[pl.ds(s * RPT + k * CD, CD)], ones_m)
        pltpu.sync_copy(ones_m, degp.at[idx_o, pl.ds(k * CD, CD)])


def _deg_pass(dst):
    out_type = jax.ShapeDtypeStruct((NW, RPT, HIDDEN), jnp.float32)
    scratch = [
        pltpu.VMEM((CD,), jnp.int32),
        pltpu.VMEM((CD, HIDDEN), jnp.float32),
        pltpu.VMEM_SHARED((NP, HIDDEN), jnp.float32),
        pltpu.SemaphoreType.DMA,
    ]
    fn = pl.kernel(
        _deg_pass_body,
        out_type=out_type,
        mesh=_mesh,
        scratch_types=scratch,
    )
    return fn(dst)


EB = 2000


def _gates_body(e_ref, *refs):
    (w10, b10, w11, b11, w12, b12, w13, b13,
     w20, b20, w21, b21, w22, b22, w23, b23, g1_ref, g2_ref) = refs
    e = e_ref[...]

    def net(ws):
        g = e
        for wr, br in ws[:-1]:
            g = jnp.maximum(jnp.dot(g, wr[...],
                                    preferred_element_type=jnp.float32)
                            + br[...], 0.0)
        wr, br = ws[-1]
        return jnp.dot(g, wr[...], preferred_element_type=jnp.float32) + br[...]

    g1_ref[...] = net([(w10, b10), (w11, b11), (w12, b12), (w13, b13)])
    g2_ref[...] = net([(w20, b20), (w21, b21), (w22, b22), (w23, b23)])


def _gates(edge, enet1, enet2):
    wspecs = []
    args = []
    for enet in (enet1, enet2):
        for lin in enet:
            wshape = lin["W"].shape
            args.append(lin["W"])
            wspecs.append(pl.BlockSpec(wshape, lambda i: (0, 0)))
            args.append(lin["b"].reshape(1, -1))
            wspecs.append(pl.BlockSpec((1, wshape[1]), lambda i: (0, 0)))
    grid = E // EB
    return pl.pallas_call(
        _gates_body,
        grid=(grid,),
        in_specs=[pl.BlockSpec((EB, EDGE_DIM), lambda i: (i, 0))] + wspecs,
        out_specs=[pl.BlockSpec((EB, HIDDEN), lambda i: (i, 0))] * 2,
        out_shape=[jax.ShapeDtypeStruct((E, HIDDEN), jnp.float32)] * 2,
    )(edge, *args)


NB = 1000


def _matmul_body(h_ref, w_ref, o_ref):
    o_ref[...] = jnp.dot(h_ref[...], w_ref[...],
                         preferred_element_type=jnp.float32)


def _matmul(h, w):
    return pl.pallas_call(
        _matmul_body,
        grid=(N // NB,),
        in_specs=[pl.BlockSpec((NB, HIDDEN), lambda i: (i, 0)),
                  pl.BlockSpec((HIDDEN, HIDDEN), lambda i: (0, 0))],
        out_specs=pl.BlockSpec((NB, HIDDEN), lambda i: (i, 0)),
        out_shape=jax.ShapeDtypeStruct((N, HIDDEN), jnp.float32),
    )(h, w)


def _comb1_body(x_ref, acc_ref, degp_ref, wr_ref, b_ref, wm_ref,
                h1_ref, hm2_ref, degc_ref):
    a = acc_ref[0] + acc_ref[1]
    dc = jnp.maximum(degp_ref[0, :, 0:1] + degp_ref[1, :, 0:1], 1.0)
    agg = a / dc
    h1 = jnp.maximum(
        jnp.dot(x_ref[...], wr_ref[...], preferred_element_type=jnp.float32)
        + agg + b_ref[...], 0.0)
    h1_ref[...] = h1
    hm2_ref[...] = jnp.dot(h1, wm_ref[...], preferred_element_type=jnp.float32)
    degc_ref[...] = dc


def _comb1(x, accp, degp, w_root, b, w_msg2):
    return pl.pallas_call(
        _comb1_body,
        grid=(N // NB,),
        in_specs=[
            pl.BlockSpec((NB, HIDDEN), lambda i: (i, 0)),
            pl.BlockSpec((NC, NB, HIDDEN), lambda i: (0, i, 0)),
            pl.BlockSpec((NC, NB, HIDDEN), lambda i: (0, i, 0)),
            pl.BlockSpec((HIDDEN, HIDDEN), lambda i: (0, 0)),
            pl.BlockSpec((1, HIDDEN), lambda i: (0, 0)),
            pl.BlockSpec((HIDDEN, HIDDEN), lambda i: (0, 0)),
        ],
        out_specs=[
            pl.BlockSpec((NB, HIDDEN), lambda i: (i, 0)),
            pl.BlockSpec((NB, HIDDEN), lambda i: (i, 0)),
            pl.BlockSpec((NB, 1), lambda i: (i, 0)),
        ],
        out_shape=[
            jax.ShapeDtypeStruct((N, HIDDEN), jnp.float32),
            jax.ShapeDtypeStruct((N, HIDDEN), jnp.float32),
            jax.ShapeDtypeStruct((N, 1), jnp.float32),
        ],
    )(x, accp, degp, w_root, b.reshape(1, -1), w_msg2)


def _tail_body(h1_ref, acc_ref, degc_ref, eps_ref,
               wr_ref, b_ref, wmu_ref, bmu_ref, wlv_ref, blv_ref,
               wx0_ref, bx0_ref, wx1_ref, bx1_ref, wfx_ref, bfx_ref,
               we0_ref, be0_ref, we1_ref, be1_ref, wfe_ref, bfe_ref,
               ox_ref, oe_ref, mu_ref, lv_ref):
    a = acc_ref[0] + acc_ref[1]
    agg = a / degc_ref[...]
    h2 = jnp.maximum(
        jnp.dot(h1_ref[...], wr_ref[...], preferred_element_type=jnp.float32)
        + agg + b_ref[...], 0.0)
    mu = jnp.clip(jnp.dot(h2, wmu_ref[...], preferred_element_type=jnp.float32)
                  + bmu_ref[...], -1.0, 1.0)
    lv = jnp.clip(jnp.dot(h2, wlv_ref[...], preferred_element_type=jnp.float32)
                  + blv_ref[...], -1.0, 1.0)
    mu_ref[...] = mu
    lv_ref[...] = lv
    z = mu + jnp.exp(0.5 * lv) * eps_ref[...]

    dx = jnp.maximum(jnp.dot(z, wx0_ref[...],
                             preferred_element_type=jnp.float32) + bx0_ref[...], 0.0)
    dx = jnp.maximum(jnp.dot(dx, wx1_ref[...],
                             preferred_element_type=jnp.float32) + bx1_ref[...], 0.0)
    ox_ref[...] = jnp.dot(dx, wfx_ref[...],
                          preferred_element_type=jnp.float32) + bfx_ref[...]

    de = jnp.maximum(jnp.dot(z, we0_ref[...],
                             preferred_element_type=jnp.float32) + be0_ref[...], 0.0)
    de = jnp.maximum(jnp.dot(de, we1_ref[...],
                             preferred_element_type=jnp.float32) + be1_ref[...], 0.0)
    oe_ref[...] = jnp.dot(de, wfe_ref[...],
                          preferred_element_type=jnp.float32) + bfe_ref[...]


def _tail(h1, accp, degc, eps, params):
    p = params
    dec_x = p["dec_x"]
    dec_e = p["dec_edge"]
    args = [
        p["ecc"][1]["W_root"], p["ecc"][1]["b"].reshape(1, -1),
        p["W_mu"], p["b_mu"].reshape(1, -1),
        p["W_lv"], p["b_lv"].reshape(1, -1),
        dec_x[0]["W"], dec_x[0]["b"].reshape(1, -1),
        dec_x[1]["W"], dec_x[1]["b"].reshape(1, -1),
        p["W_fx"], p["b_fx"].reshape(1, -1),
        dec_e[0]["W"], dec_e[0]["b"].reshape(1, -1),
        dec_e[1]["W"], dec_e[1]["b"].reshape(1, -1),
        p["W_fe"], p["b_fe"].reshape(1, -1),
    ]
    wspecs = [pl.BlockSpec(a.shape, lambda i: (0, 0)) for a in args]
    return pl.pallas_call(
        _tail_body,
        grid=(N // NB,),
        in_specs=[
            pl.BlockSpec((NB, HIDDEN), lambda i: (i, 0)),
            pl.BlockSpec((NC, NB, HIDDEN), lambda i: (0, i, 0)),
            pl.BlockSpec((NB, 1), lambda i: (i, 0)),
            pl.BlockSpec((NB, LATENT), lambda i: (i, 0)),
        ] + wspecs,
        out_specs=[
            pl.BlockSpec((NB, FX_DIM), lambda i: (i, 0)),
            pl.BlockSpec((NB, FE_DIM), lambda i: (i, 0)),
            pl.BlockSpec((NB, LATENT), lambda i: (i, 0)),
            pl.BlockSpec((NB, LATENT), lambda i: (i, 0)),
        ],
        out_shape=[
            jax.ShapeDtypeStruct((N, FX_DIM), jnp.float32),
            jax.ShapeDtypeStruct((N, FE_DIM), jnp.float32),
            jax.ShapeDtypeStruct((N, LATENT), jnp.float32),
            jax.ShapeDtypeStruct((N, LATENT), jnp.float32),
        ],
    )(h1, accp, degc, eps, *args)


def kernel(x, adj, edge, params):
    src = adj[0]
    dst = adj[1]
    eps = jax.random.normal(jax.random.key(42), (N, LATENT), dtype=jnp.float32)

    ecc1, ecc2 = params["ecc"]
    g1, g2 = _gates(edge, ecc1["edge_net"], ecc2["edge_net"])

    hm1 = _matmul(x, ecc1["W_msg"])
    degp = _deg_pass(dst).reshape(NC, NP, HIDDEN)
    accp1 = _sc_pass(hm1, g1, src, dst).reshape(NC, NP, HIDDEN)
    h1, hm2, degc = _comb1(x, accp1, degp, ecc1["W_root"], ecc1["b"],
                           ecc2["W_msg"])
    accp2 = _sc_pass(hm2, g2, src, dst).reshape(NC, NP, HIDDEN)
    out_x, oe, mu, lv = _tail(h1, accp2, degc, eps, params)
    return (out_x, oe.reshape(N, MAX_SIZE, EDGE_DIM + EDGE_CLASS - 1), mu, lv)


# speedup vs baseline: 3.4936x; 1.1222x over previous
import functools

import jax
import jax.numpy as jnp
from jax import lax
from jax.experimental import pallas as pl
from jax.experimental.pallas import tpu as pltpu
from jax.experimental.pallas import tpu_sc as plsc

N = 10000
E = 320000
X_DIM = 128
EDGE_DIM = 16
HIDDEN = 128
LATENT = 64
AA_DIM = 20
SS_DIM = 7
X_CLASS = 2
EDGE_CLASS = 3
MAX_SIZE = 30
FX_DIM = AA_DIM * SS_DIM + X_DIM - X_CLASS
FE_DIM = MAX_SIZE * (EDGE_DIM + EDGE_CLASS - 1)

NC = 2
NS = 16
NW = NC * NS
EP = E // NW
C = 80
NCH = EP // C
CD = 80
NCHD = EP // CD
NP = 10240
RPT = NP // NS

_mesh = plsc.VectorSubcoreMesh(
    core_axis_name="c", subcore_axis_name="s", num_cores=NC, num_subcores=NS)


def _sc_pass_body(hm, gate, adj, accp,
                  idx_s0, idx_s1, idx_d0, idx_d1, rows0, rows1, gv0, gv1,
                  zb, acc_sh,
                  isem0, isem1, gsem0, gsem1, lsem0, lsem1):
    idx_s = (idx_s0, idx_s1)
    idx_d = (idx_d0, idx_d1)
    rows = (rows0, rows1)
    gv = (gv0, gv1)
    isem = (isem0, isem1)
    gsem = (gsem0, gsem1)
    lsem = (lsem0, lsem1)
    c = lax.axis_index("c")
    s = lax.axis_index("s")
    w = s * NC + c

    zeros = jnp.zeros((16,), jnp.float32)

    for i in range(16):
        for j in range(HIDDEN // 16):
            zb[i, pl.ds(j * 16, 16)] = zeros

    @pl.loop(0, RPT // 16)
    def _zacc(t):
        pltpu.sync_copy(zb, acc_sh.at[pl.ds(s * RPT + t * 16, 16)])

    plsc.subcore_barrier()

    ebase = w * EP

    def cbase(j):
        return ebase + jnp.where(j < NCH, j, 0) * C

    def start_idx(j, b):
        pltpu.async_copy(adj.at[pl.ds(cbase(j), C)], idx_s[b], isem[b])
        pltpu.async_copy(adj.at[pl.ds(E + cbase(j), C)], idx_d[b], isem[b])

    def wait_idx(b):
        pltpu.make_async_copy(adj.at[pl.ds(0, C)], idx_s[b], isem[b]).wait()
        pltpu.make_async_copy(adj.at[pl.ds(0, C)], idx_d[b], isem[b]).wait()

    def start_fetch(j, b):
        pltpu.async_copy(hm.at[idx_s[b]], rows[b], gsem[b])
        pltpu.async_copy(gate.at[pl.ds(cbase(j), C)], gv[b], lsem[b])

    def wait_fetch(b):
        pltpu.make_async_copy(hm.at[idx_s[b]], rows[b], gsem[b]).wait()
        pltpu.make_async_copy(gate.at[pl.ds(0, C)], gv[b], lsem[b]).wait()

    start_idx(0, 0)
    start_idx(1, 1)
    wait_idx(0)
    start_fetch(0, 0)

    def _mul(b):
        def _mrow(r, cy):
            for k2 in range(HIDDEN // 16):
                sl = pl.ds(k2 * 16, 16)
                rows[b][r, sl] = rows[b][r, sl] * gv[b][r, sl]
            return cy
        lax.fori_loop(0, C, _mrow, 0)

    @pl.loop(0, NCH - 1, step=2)
    def _chunk(i):
        for b in range(2):
            j = i + b
            o = b ^ 1
            wait_idx(o)
            start_fetch(j + 1, o)
            wait_fetch(b)
            _mul(b)
            pltpu.sync_copy(rows[b], acc_sh.at[idx_d[b]], add=True)
            start_idx(j + 2, b)

    wait_fetch(0)
    _mul(0)
    pltpu.sync_copy(rows[0], acc_sh.at[idx_d[0]], add=True)
    wait_idx(1)

    plsc.subcore_barrier()

    idx_o = c * NS + s
    for k in range(RPT // C):
        b = k % 2
        if k >= 2:
            pltpu.make_async_copy(rows[b], accp.at[idx_o, pl.ds(0, C)],
                                  gsem[b]).wait()
        pltpu.sync_copy(acc_sh.at[pl.ds(s * RPT + k * C, C)], rows[b])
        pltpu.async_copy(rows[b], accp.at[idx_o, pl.ds(k * C, C)], gsem[b])
    pltpu.make_async_copy(rows[0], accp.at[idx_o, pl.ds(0, C)], gsem[0]).wait()
    pltpu.make_async_copy(rows[1], accp.at[idx_o, pl.ds(0, C)], gsem[1]).wait()


def _sc_pass(hm, gate, adj):
    out_type = jax.ShapeDtypeStruct((NW, RPT, HIDDEN), jnp.float32)
    scratch = [
        pltpu.VMEM((C,), jnp.int32),
        pltpu.VMEM((C,), jnp.int32),
        pltpu.VMEM((C,), jnp.int32),
        pltpu.VMEM((C,), jnp.int32),
        pltpu.VMEM((C, HIDDEN), jnp.float32),
        pltpu.VMEM((C, HIDDEN), jnp.float32),
        pltpu.VMEM((C, HIDDEN), jnp.float32),
        pltpu.VMEM((C, HIDDEN), jnp.float32),
        pltpu.VMEM((16, HIDDEN), jnp.float32),
        pltpu.VMEM_SHARED((NP, HIDDEN), jnp.float32),
        pltpu.SemaphoreType.DMA,
        pltpu.SemaphoreType.DMA,
        pltpu.SemaphoreType.DMA,
        pltpu.SemaphoreType.DMA,
        pltpu.SemaphoreType.DMA,
        pltpu.SemaphoreType.DMA,
    ]
    fn = pl.kernel(
        _sc_pass_body,
        out_type=out_type,
        mesh=_mesh,
        scratch_types=scratch,
    )
    return fn(hm, gate, adj)


def _deg_pass_body(adj, degp, idx_d, ones_m, deg_sh, gsem):
    c = lax.axis_index("c")
    s = lax.axis_index("s")
    w = s * NC + c

    ones = jnp.ones((16,), jnp.float32)
    zeros = jnp.zeros((16,), jnp.float32)

    def _fones(r, carry):
        for j in range(HIDDEN // 16):
            ones_m[r, pl.ds(j * 16, 16)] = zeros
        return carry
    lax.fori_loop(0, CD, _fones, 0)

    @pl.loop(0, RPT // CD)
    def _zdeg(t):
        pltpu.sync_copy(ones_m, deg_sh.at[pl.ds(s * RPT + t * CD, CD)])

    def _fones2(r, carry):
        for j in range(HIDDEN // 16):
            ones_m[r, pl.ds(j * 16, 16)] = ones
        return carry
    lax.fori_loop(0, CD, _fones2, 0)

    plsc.subcore_barrier()

    @pl.loop(0, NCHD)
    def _chunk(i):
        base = w * EP + i * CD
        pltpu.sync_copy(adj.at[pl.ds(E + base, CD)], idx_d)
        pltpu.sync_copy(ones_m, deg_sh.at[idx_d], add=True)

    plsc.subcore_barrier()

    idx_o = c * NS + s
    for k in range(RPT // CD):
        pltpu.sync_copy(deg_sh.at[pl.ds(s * RPT + k * CD, CD)], ones_m)
        pltpu.sync_copy(ones_m, degp.at[idx_o, pl.ds(k * CD, CD)])


def _deg_pass(adj):
    out_type = jax.ShapeDtypeStruct((NW, RPT, HIDDEN), jnp.float32)
    scratch = [
        pltpu.VMEM((CD,), jnp.int32),
        pltpu.VMEM((CD, HIDDEN), jnp.float32),
        pltpu.VMEM_SHARED((NP, HIDDEN), jnp.float32),
        pltpu.SemaphoreType.DMA,
    ]
    fn = pl.kernel(
        _deg_pass_body,
        out_type=out_type,
        mesh=_mesh,
        scratch_types=scratch,
    )
    return fn(adj)


EB = 2000


def _gates_body(e_ref, *refs):
    (w10, b10, w11, b11, w12, b12, w13, b13,
     w20, b20, w21, b21, w22, b22, w23, b23, g1_ref, g2_ref) = refs
    e = e_ref[...]

    def net(ws):
        g = e
        for wr, br in ws[:-1]:
            g = jnp.maximum(jnp.dot(g, wr[...],
                                    preferred_element_type=jnp.float32)
                            + br[...], 0.0)
        wr, br = ws[-1]
        return jnp.dot(g, wr[...], preferred_element_type=jnp.float32) + br[...]

    g1_ref[...] = net([(w10, b10), (w11, b11), (w12, b12), (w13, b13)])
    g2_ref[...] = net([(w20, b20), (w21, b21), (w22, b22), (w23, b23)])


def _gates(edge, enet1, enet2):
    wspecs = []
    args = []
    for enet in (enet1, enet2):
        for lin in enet:
            wshape = lin["W"].shape
            args.append(lin["W"])
            wspecs.append(pl.BlockSpec(wshape, lambda i: (0, 0)))
            args.append(lin["b"].reshape(1, -1))
            wspecs.append(pl.BlockSpec((1, wshape[1]), lambda i: (0, 0)))
    grid = E // EB
    return pl.pallas_call(
        _gates_body,
        grid=(grid,),
        in_specs=[pl.BlockSpec((EB, EDGE_DIM), lambda i: (i, 0))] + wspecs,
        out_specs=[pl.BlockSpec((EB, HIDDEN), lambda i: (i, 0))] * 2,
        out_shape=[jax.ShapeDtypeStruct((E, HIDDEN), jnp.float32)] * 2,
    )(edge, *args)


NB = 1000


def _matmul_body(h_ref, w_ref, o_ref):
    o_ref[...] = jnp.dot(h_ref[...], w_ref[...],
                         preferred_element_type=jnp.float32)


def _matmul(h, w):
    return pl.pallas_call(
        _matmul_body,
        grid=(N // NB,),
        in_specs=[pl.BlockSpec((NB, HIDDEN), lambda i: (i, 0)),
                  pl.BlockSpec((HIDDEN, HIDDEN), lambda i: (0, 0))],
        out_specs=pl.BlockSpec((NB, HIDDEN), lambda i: (i, 0)),
        out_shape=jax.ShapeDtypeStruct((N, HIDDEN), jnp.float32),
    )(h, w)


def _comb1_body(x_ref, acc_ref, degp_ref, wr_ref, b_ref, wm_ref,
                h1_ref, hm2_ref, degc_ref):
    a = acc_ref[0] + acc_ref[1]
    dc = jnp.maximum(degp_ref[0, :, 0:1] + degp_ref[1, :, 0:1], 1.0)
    agg = a / dc
    h1 = jnp.maximum(
        jnp.dot(x_ref[...], wr_ref[...], preferred_element_type=jnp.float32)
        + agg + b_ref[...], 0.0)
    h1_ref[...] = h1
    hm2_ref[...] = jnp.dot(h1, wm_ref[...], preferred_element_type=jnp.float32)
    degc_ref[...] = dc


def _comb1(x, accp, degp, w_root, b, w_msg2):
    return pl.pallas_call(
        _comb1_body,
        grid=(N // NB,),
        in_specs=[
            pl.BlockSpec((NB, HIDDEN), lambda i: (i, 0)),
            pl.BlockSpec((NC, NB, HIDDEN), lambda i: (0, i, 0)),
            pl.BlockSpec((NC, NB, HIDDEN), lambda i: (0, i, 0)),
            pl.BlockSpec((HIDDEN, HIDDEN), lambda i: (0, 0)),
            pl.BlockSpec((1, HIDDEN), lambda i: (0, 0)),
            pl.BlockSpec((HIDDEN, HIDDEN), lambda i: (0, 0)),
        ],
        out_specs=[
            pl.BlockSpec((NB, HIDDEN), lambda i: (i, 0)),
            pl.BlockSpec((NB, HIDDEN), lambda i: (i, 0)),
            pl.BlockSpec((NB, 1), lambda i: (i, 0)),
        ],
        out_shape=[
            jax.ShapeDtypeStruct((N, HIDDEN), jnp.float32),
            jax.ShapeDtypeStruct((N, HIDDEN), jnp.float32),
            jax.ShapeDtypeStruct((N, 1), jnp.float32),
        ],
    )(x, accp, degp, w_root, b.reshape(1, -1), w_msg2)


def _tail_body(h1_ref, acc_ref, degc_ref, eps_ref,
               wr_ref, b_ref, wmu_ref, bmu_ref, wlv_ref, blv_ref,
               wx0_ref, bx0_ref, wx1_ref, bx1_ref, wfx_ref, bfx_ref,
               we0_ref, be0_ref, we1_ref, be1_ref, wfe_ref, bfe_ref,
               ox_ref, oe_ref, mu_ref, lv_ref):
    a = acc_ref[0] + acc_ref[1]
    agg = a / degc_ref[...]
    h2 = jnp.maximum(
        jnp.dot(h1_ref[...], wr_ref[...], preferred_element_type=jnp.float32)
        + agg + b_ref[...], 0.0)
    mu = jnp.clip(jnp.dot(h2, wmu_ref[...], preferred_element_type=jnp.float32)
                  + bmu_ref[...], -1.0, 1.0)
    lv = jnp.clip(jnp.dot(h2, wlv_ref[...], preferred_element_type=jnp.float32)
                  + blv_ref[...], -1.0, 1.0)
    mu_ref[...] = mu
    lv_ref[...] = lv
    z = mu + jnp.exp(0.5 * lv) * eps_ref[...]

    dx = jnp.maximum(jnp.dot(z, wx0_ref[...],
                             preferred_element_type=jnp.float32) + bx0_ref[...], 0.0)
    dx = jnp.maximum(jnp.dot(dx, wx1_ref[...],
                             preferred_element_type=jnp.float32) + bx1_ref[...], 0.0)
    ox_ref[...] = jnp.dot(dx, wfx_ref[...],
                          preferred_element_type=jnp.float32) + bfx_ref[...]

    de = jnp.maximum(jnp.dot(z, we0_ref[...],
                             preferred_element_type=jnp.float32) + be0_ref[...], 0.0)
    de = jnp.maximum(jnp.dot(de, we1_ref[...],
                             preferred_element_type=jnp.float32) + be1_ref[...], 0.0)
    oe_ref[...] = jnp.dot(de, wfe_ref[...],
                          preferred_element_type=jnp.float32) + bfe_ref[...]


def _tail(h1, accp, degc, eps, params):
    p = params
    dec_x = p["dec_x"]
    dec_e = p["dec_edge"]
    args = [
        p["ecc"][1]["W_root"], p["ecc"][1]["b"].reshape(1, -1),
        p["W_mu"], p["b_mu"].reshape(1, -1),
        p["W_lv"], p["b_lv"].reshape(1, -1),
        dec_x[0]["W"], dec_x[0]["b"].reshape(1, -1),
        dec_x[1]["W"], dec_x[1]["b"].reshape(1, -1),
        p["W_fx"], p["b_fx"].reshape(1, -1),
        dec_e[0]["W"], dec_e[0]["b"].reshape(1, -1),
        dec_e[1]["W"], dec_e[1]["b"].reshape(1, -1),
        p["W_fe"], p["b_fe"].reshape(1, -1),
    ]
    wspecs = [pl.BlockSpec(a.shape, lambda i: (0, 0)) for a in args]
    return pl.pallas_call(
        _tail_body,
        grid=(N // NB,),
        in_specs=[
            pl.BlockSpec((NB, HIDDEN), lambda i: (i, 0)),
            pl.BlockSpec((NC, NB, HIDDEN), lambda i: (0, i, 0)),
            pl.BlockSpec((NB, 1), lambda i: (i, 0)),
            pl.BlockSpec((NB, LATENT), lambda i: (i, 0)),
        ] + wspecs,
        out_specs=[
            pl.BlockSpec((NB, FX_DIM), lambda i: (i, 0)),
            pl.BlockSpec((NB, FE_DIM), lambda i: (i, 0)),
            pl.BlockSpec((NB, LATENT), lambda i: (i, 0)),
            pl.BlockSpec((NB, LATENT), lambda i: (i, 0)),
        ],
        out_shape=[
            jax.ShapeDtypeStruct((N, FX_DIM), jnp.float32),
            jax.ShapeDtypeStruct((N, FE_DIM), jnp.float32),
            jax.ShapeDtypeStruct((N, LATENT), jnp.float32),
            jax.ShapeDtypeStruct((N, LATENT), jnp.float32),
        ],
    )(h1, accp, degc, eps, *args)


def kernel(x, adj, edge, params):
    adj = adj.reshape(2 * E)
    eps = jax.random.normal(jax.random.key(42), (N, LATENT), dtype=jnp.float32)

    ecc1, ecc2 = params["ecc"]
    g1, g2 = _gates(edge, ecc1["edge_net"], ecc2["edge_net"])

    hm1 = _matmul(x, ecc1["W_msg"])
    degp = _deg_pass(adj).reshape(NC, NP, HIDDEN)
    accp1 = _sc_pass(hm1, g1, adj).reshape(NC, NP, HIDDEN)
    h1, hm2, degc = _comb1(x, accp1, degp, ecc1["W_root"], ecc1["b"],
                           ecc2["W_msg"])
    accp2 = _sc_pass(hm2, g2, adj).reshape(NC, NP, HIDDEN)
    out_x, oe, mu, lv = _tail(h1, accp2, degc, eps, params)
    return (out_x, oe.reshape(N, MAX_SIZE, EDGE_DIM + EDGE_CLASS - 1), mu, lv)
